# baseline (speedup 1.0000x reference)
import functools

import jax
import jax.numpy as jnp
from jax import lax
from jax.experimental import pallas as pl
from jax.experimental.pallas import tpu as pltpu
from jax.experimental.pallas import tpu_sc as plsc

B, H, W, C = 8, 112, 112, 96
P_IN = H * W
P_OUT = 4 * P_IN
CT = 16
NG = C // CT
N_JOBS = B * NG
NC, NS = 2, 16
JOBS_PER = N_JOBS // NC
POS_T = P_IN // NS
CHUNK = POS_T * CT // 128
DEPTH = 8
ACC = P_OUT * CT
ACC_T = ACC // NS
NRB = 32
ZCH = ACC_T // NRB
RROWS = ZCH // CT

_mesh = plsc.VectorSubcoreMesh(
    core_axis_name="c", subcore_axis_name="s", num_cores=NC, num_subcores=NS
)


@functools.partial(
    pl.kernel,
    out_type=jax.ShapeDtypeStruct((B, P_OUT, C), jnp.float32),
    mesh=_mesh,
    scratch_types=[
        pltpu.VMEM((POS_T, CT), jnp.int32),
        pltpu.VMEM((POS_T, CT), jnp.float32),
        pltpu.VMEM((CHUNK, 128), jnp.int32),
        pltpu.VMEM((CHUNK, 128), jnp.float32),
        pltpu.VMEM((2, ZCH), jnp.float32),
        pltpu.VMEM((2, RROWS, CT), jnp.float32),
        pltpu.VMEM((ZCH,), jnp.float32),
        pltpu.VMEM_SHARED((ACC,), jnp.float32),
        pltpu.SemaphoreType.DMA,
        pltpu.SemaphoreType.DMA,
        pltpu.SemaphoreType.DMA,
        pltpu.SemaphoreType.DMA,
        pltpu.SemaphoreType.DMA,
    ],
    compiler_params=pltpu.CompilerParams(use_tc_tiling_on_sc=False),
)
def _unpool_sc(upd_hbm, mask_hbm, out_hbm, mask_v, upd_v, idx_v, val_v,
               tmp_v, t2d_v, zero_v, acc_sh, sem_s, sem_in, sem_rb, sem_out,
               sem_z):
    cid = lax.axis_index("c")
    sid = lax.axis_index("s")
    dc0 = lax.iota(jnp.int32, 16)
    third = jnp.float32(1.0 / 3.0)

    tb = sid * ACC_T
    rb = sid * (P_OUT // NS)

    def in_desc(k):
        g = k * NC + cid
        b = g // NG
        c0 = (g % NG) * CT
        src_m = mask_hbm.at[b, pl.ds(sid * POS_T, POS_T), pl.ds(c0, CT)]
        src_u = upd_hbm.at[b, pl.ds(sid * POS_T, POS_T), pl.ds(c0, CT)]
        return (
            pltpu.make_async_copy(src_m, mask_v, sem_in),
            pltpu.make_async_copy(src_u, upd_v, sem_in),
        )

    def scat_desc(j):
        return pltpu.make_async_copy(
            val_v.at[j], acc_sh.at[idx_v.at[j]], sem_s
        )

    def zinit(i, carry):
        zero_v[pl.ds(i * 16, 16)] = jnp.zeros((16,), jnp.float32)
        return carry

    lax.fori_loop(0, ZCH // 16, zinit, 0)

    for d in in_desc(0):
        d.start()

    def zslice(i, c):
        pltpu.async_copy(zero_v, acc_sh.at[pl.ds(tb + i * ZCH, ZCH)], sem_z)
        return c

    lax.fori_loop(0, NRB, zslice, 0)

    def zdrain(i, c):
        pltpu.make_async_copy(
            zero_v, acc_sh.at[pl.ds(tb + i * ZCH, ZCH)], sem_z
        ).wait()
        return c

    lax.fori_loop(0, NRB, zdrain, 0)
    plsc.subcore_barrier()

    def job_body(k, carry):
        g = k * NC + cid
        b = g // NG
        c0 = (g % NG) * CT
        for d in in_desc(k):
            d.wait()

        def chunk_body(j, c):
            @pl.when(j >= DEPTH)
            def _():
                scat_desc(j - DEPTH).wait()

            for q in range(8):
                pos = j * 8 + q
                m = mask_v[pos, :]
                t1 = jnp.right_shift(m, 5)
                qi = (t1.astype(jnp.float32) * third).astype(jnp.int32)
                r = t1 - (qi + qi + qi)
                qi = jnp.where(r >= 3, qi + 1, qi)
                a = jnp.left_shift(qi, 4) + dc0
                idx_v[j, pl.ds(q * 16, 16)] = a
                val_v[j, pl.ds(q * 16, 16)] = upd_v[pos, :]
            pltpu.async_copy(
                val_v.at[j], acc_sh.at[idx_v.at[j]], sem_s, add=True
            )
            return c

        lax.fori_loop(0, CHUNK, chunk_body, 0)

        def sdrain(j, c):
            scat_desc(CHUNK - DEPTH + j).wait()
            return c

        lax.fori_loop(0, DEPTH, sdrain, 0)

        @pl.when(k + 1 < JOBS_PER)
        def _():
            for d in in_desc(k + 1):
                d.start()

        plsc.subcore_barrier()

        def rb_in(i, buf):
            return pltpu.make_async_copy(
                acc_sh.at[pl.ds(tb + i * ZCH, ZCH)], tmp_v.at[buf], sem_rb
            )

        def rb_out(i, buf):
            return pltpu.make_async_copy(
                t2d_v.at[buf],
                out_hbm.at[b, pl.ds(rb + i * RROWS, RROWS), pl.ds(c0, CT)],
                sem_out,
            )

        rb_in(0, 0).start()

        def rb_body(i, c):
            buf = jnp.bitwise_and(i, 1)

            @pl.when(i + 1 < NRB)
            def _():
                rb_in(i + 1, 1 - buf).start()

            rb_in(i, buf).wait()
            pltpu.async_copy(
                zero_v, acc_sh.at[pl.ds(tb + i * ZCH, ZCH)], sem_z
            )

            @pl.when(i >= 2)
            def _():
                rb_out(i - 2, buf).wait()

            def cp(rr, cc):
                t2d_v[buf, rr, :] = tmp_v[buf, pl.ds(rr * CT, CT)]
                return cc

            lax.fori_loop(0, RROWS, cp, 0)
            rb_out(i, buf).start()
            return c

        lax.fori_loop(0, NRB, rb_body, 0)
        rb_out(NRB - 2, 0).wait()
        rb_out(NRB - 1, 1).wait()
        lax.fori_loop(0, NRB, zdrain, 0)
        plsc.subcore_barrier()
        return carry

    lax.fori_loop(0, JOBS_PER, job_body, 0)


@jax.jit
def kernel(updates, mask):
    upd = updates.reshape(B, P_IN, C)
    msk = mask.astype(jnp.int32).reshape(B, P_IN, C)
    out = _unpool_sc(upd, msk)
    return out.reshape(B, 2 * H, 2 * W, C)


# speedup vs baseline: 1.0028x; 1.0028x over previous
import functools

import jax
import jax.numpy as jnp
from jax import lax
from jax.experimental import pallas as pl
from jax.experimental.pallas import tpu as pltpu
from jax.experimental.pallas import tpu_sc as plsc

B, H, W, C = 8, 112, 112, 96
P_IN = H * W
P_OUT = 4 * P_IN
CT = 16
NG = C // CT
N_JOBS = B * NG
NC, NS = 2, 16
JOBS_PER = N_JOBS // NC
POS_T = P_IN // NS
CHUNK = POS_T * CT // 128
DEPTH = 8
ACC = P_OUT * CT
ACC_T = ACC // NS
NRB = 16
ZCH = ACC_T // NRB
RROWS = ZCH // CT

_mesh = plsc.VectorSubcoreMesh(
    core_axis_name="c", subcore_axis_name="s", num_cores=NC, num_subcores=NS
)


@functools.partial(
    pl.kernel,
    out_type=jax.ShapeDtypeStruct((B, P_OUT, C), jnp.float32),
    mesh=_mesh,
    scratch_types=[
        pltpu.VMEM((POS_T, CT), jnp.int32),
        pltpu.VMEM((POS_T, CT), jnp.float32),
        pltpu.VMEM((CHUNK, 128), jnp.int32),
        pltpu.VMEM((CHUNK, 128), jnp.float32),
        pltpu.VMEM((2, ZCH), jnp.float32),
        pltpu.VMEM((2, RROWS, CT), jnp.float32),
        pltpu.VMEM((ZCH,), jnp.float32),
        pltpu.VMEM_SHARED((ACC,), jnp.float32),
        pltpu.SemaphoreType.DMA,
        pltpu.SemaphoreType.DMA,
        pltpu.SemaphoreType.DMA,
        pltpu.SemaphoreType.DMA,
        pltpu.SemaphoreType.DMA,
    ],
    compiler_params=pltpu.CompilerParams(use_tc_tiling_on_sc=False),
)
def _unpool_sc(upd_hbm, mask_hbm, out_hbm, mask_v, upd_v, idx_v, val_v,
               tmp_v, t2d_v, zero_v, acc_sh, sem_s, sem_in, sem_rb, sem_out,
               sem_z):
    cid = lax.axis_index("c")
    sid = lax.axis_index("s")
    dc0 = lax.iota(jnp.int32, 16)
    third = jnp.float32(1.0 / 3.0)

    tb = sid * ACC_T
    rb = sid * (P_OUT // NS)

    def in_desc(k):
        g = k * NC + cid
        b = g // NG
        c0 = (g % NG) * CT
        src_m = mask_hbm.at[b, pl.ds(sid * POS_T, POS_T), pl.ds(c0, CT)]
        src_u = upd_hbm.at[b, pl.ds(sid * POS_T, POS_T), pl.ds(c0, CT)]
        return (
            pltpu.make_async_copy(src_m, mask_v, sem_in),
            pltpu.make_async_copy(src_u, upd_v, sem_in),
        )

    def scat_desc(j):
        return pltpu.make_async_copy(
            val_v.at[j], acc_sh.at[idx_v.at[j]], sem_s
        )

    def zinit(i, carry):
        zero_v[pl.ds(i * 16, 16)] = jnp.zeros((16,), jnp.float32)
        return carry

    lax.fori_loop(0, ZCH // 16, zinit, 0)

    for d in in_desc(0):
        d.start()

    def zslice(i, c):
        pltpu.async_copy(zero_v, acc_sh.at[pl.ds(tb + i * ZCH, ZCH)], sem_z)
        return c

    lax.fori_loop(0, NRB, zslice, 0)

    def zdrain(i, c):
        pltpu.make_async_copy(
            zero_v, acc_sh.at[pl.ds(tb + i * ZCH, ZCH)], sem_z
        ).wait()
        return c

    lax.fori_loop(0, NRB, zdrain, 0)
    plsc.subcore_barrier()

    def job_body(k, carry):
        g = k * NC + cid
        b = g // NG
        c0 = (g % NG) * CT
        for d in in_desc(k):
            d.wait()

        def chunk_body(j, c):
            @pl.when(j >= DEPTH)
            def _():
                scat_desc(j - DEPTH).wait()

            for q in range(8):
                pos = j * 8 + q
                m = mask_v[pos, :]
                t1 = jnp.right_shift(m, 5)
                qi = (t1.astype(jnp.float32) * third).astype(jnp.int32)
                r = t1 - (qi + qi + qi)
                qi = jnp.where(r >= 3, qi + 1, qi)
                a = jnp.left_shift(qi, 4) + dc0
                idx_v[j, pl.ds(q * 16, 16)] = a
                val_v[j, pl.ds(q * 16, 16)] = upd_v[pos, :]
            pltpu.async_copy(
                val_v.at[j], acc_sh.at[idx_v.at[j]], sem_s, add=True
            )
            return c

        lax.fori_loop(0, CHUNK, chunk_body, 0)

        def sdrain(j, c):
            scat_desc(CHUNK - DEPTH + j).wait()
            return c

        lax.fori_loop(0, DEPTH, sdrain, 0)

        @pl.when(k + 1 < JOBS_PER)
        def _():
            for d in in_desc(k + 1):
                d.start()

        plsc.subcore_barrier()

        def rb_in(i, buf):
            return pltpu.make_async_copy(
                acc_sh.at[pl.ds(tb + i * ZCH, ZCH)], tmp_v.at[buf], sem_rb
            )

        def rb_out(i, buf):
            return pltpu.make_async_copy(
                t2d_v.at[buf],
                out_hbm.at[b, pl.ds(rb + i * RROWS, RROWS), pl.ds(c0, CT)],
                sem_out,
            )

        rb_in(0, 0).start()

        def rb_body(i, c):
            buf = jnp.bitwise_and(i, 1)

            @pl.when(i + 1 < NRB)
            def _():
                rb_in(i + 1, 1 - buf).start()

            rb_in(i, buf).wait()
            pltpu.async_copy(
                zero_v, acc_sh.at[pl.ds(tb + i * ZCH, ZCH)], sem_z
            )

            @pl.when(i >= 2)
            def _():
                rb_out(i - 2, buf).wait()

            def cp(rr, cc):
                t2d_v[buf, rr, :] = tmp_v[buf, pl.ds(rr * CT, CT)]
                return cc

            lax.fori_loop(0, RROWS, cp, 0)
            rb_out(i, buf).start()
            return c

        lax.fori_loop(0, NRB, rb_body, 0)
        rb_out(NRB - 2, 0).wait()
        rb_out(NRB - 1, 1).wait()
        lax.fori_loop(0, NRB, zdrain, 0)
        plsc.subcore_barrier()
        return carry

    lax.fori_loop(0, JOBS_PER, job_body, 0)


@jax.jit
def kernel(updates, mask):
    upd = updates.reshape(B, P_IN, C)
    msk = mask.astype(jnp.int32).reshape(B, P_IN, C)
    out = _unpool_sc(upd, msk)
    return out.reshape(B, 2 * H, 2 * W, C)
